# baseline (speedup 1.0000x reference)
import functools

import jax
import jax.numpy as jnp
from jax import lax
from jax.experimental import pallas as pl
from jax.experimental.pallas import tpu as pltpu
from jax.experimental.pallas import tpu_sc as plsc

_VOCAB = 1000000
_EMBED = 64
_IMG = 128
_B = 16384
_L = 200

_NC = 2
_NS = 16
_NW = _NC * _NS
_SPW = _B // _NW
_HALF = _SPW // 2
_IDX_PITCH = 256
_IDX_SPLIT = (96, 104)
_QTR = 128
_LANES = 16
_CHUNKS = _EMBED // _LANES
_PACKED_W = _EMBED // 2

_SB = 4096
_SB_LOG2 = _SB.bit_length() - 1
_NBLK = (_VOCAB + 4 * _SB - 1) // (4 * _SB)
_PACKED_ROWS = _NBLK * _SB


def _to_bf16_hi(bits):
  return (bits + 0x7FFF + ((bits >> 16) & 1)) >> 16


def _tc_detile(table_t):
  def body(x0_ref, x1_ref, x2_ref, x3_ref, o_ref):
    eye = jnp.eye(_PACKED_W, dtype=jnp.float32)
    for j, x_ref in enumerate((x0_ref, x1_ref, x2_ref, x3_ref)):
      x = x_ref[...]
      lo = lax.bitcast_convert_type(x[:_PACKED_W, :], jnp.int32)
      hi = lax.bitcast_convert_type(x[_PACKED_W:, :], jnp.int32)
      w = (_to_bf16_hi(hi) << 16) | (_to_bf16_hi(lo) & 0xFFFF)
      wf = lax.bitcast_convert_type(w, jnp.float32)
      o_ref[:, pl.ds(j * _PACKED_W, _PACKED_W)] = lax.dot_general(
          wf, eye, (((0,), (0,)), ((), ())),
          preferred_element_type=jnp.float32)

  nb_in = (_VOCAB + _SB - 1) // _SB - 1

  def imap(j):
    return lambda i: (0, jnp.minimum(4 * i + j, nb_in))

  return pl.pallas_call(
      body,
      grid=(_NBLK,),
      in_specs=[pl.BlockSpec((_EMBED, _SB), imap(j)) for j in range(4)],
      out_specs=pl.BlockSpec((_SB, 4 * _PACKED_W), lambda i: (i, 0)),
      out_shape=jax.ShapeDtypeStruct((_PACKED_ROWS, 4 * _PACKED_W),
                                     jnp.float32),
  )(table_t, table_t, table_t, table_t)


def _tc_idx_xform(text):
  blk = 2048

  def body(x_ref, o_ref):
    v = x_ref[...]
    r = ((v & ~(4 * _SB - 1)) + ((v & (_SB - 1)) << 2)
         + ((v >> _SB_LOG2) & 3))
    pad = jnp.zeros((blk, _IDX_PITCH - _L), jnp.int32)
    o_ref[...] = jnp.concatenate([r, pad], axis=1)

  return pl.pallas_call(
      body,
      grid=(_B // blk,),
      in_specs=[pl.BlockSpec((blk, _L), lambda i: (i, 0))],
      out_specs=pl.BlockSpec((blk, _IDX_PITCH), lambda i: (i, 0)),
      out_shape=jax.ShapeDtypeStruct((_B, _IDX_PITCH), jnp.int32),
  )(text)


def _sc_pool(table, idx2):
  mesh = plsc.VectorSubcoreMesh(core_axis_name="c", subcore_axis_name="s")

  @functools.partial(
      pl.kernel,
      out_type=jax.ShapeDtypeStruct((_B, _EMBED), jnp.float32),
      mesh=mesh,
      compiler_params=pltpu.CompilerParams(use_tc_tiling_on_sc=False,
                                           needs_layout_passes=False),
      scratch_types=[
          pltpu.VMEM((_QTR, _IDX_PITCH), jnp.int32),
          pltpu.VMEM((2, _L, _PACKED_W), jnp.float32),
          pltpu.VMEM((_QTR, _EMBED), jnp.float32),
          pltpu.SemaphoreType.DMA,
          pltpu.SemaphoreType.DMA,
      ],
  )
  def pool(table_hbm, idx_hbm, out_hbm, idx_v, rows_v, out_v, sem0, sem1):
    wid = lax.axis_index("s") * _NC + lax.axis_index("c")
    sems = (sem0, sem1)

    def descr(buf, s_loc, j):
      off = j * _IDX_SPLIT[0]
      return pltpu.make_async_copy(
          table_hbm.at[idx_v.at[s_loc, pl.ds(off, _IDX_SPLIT[j])]],
          rows_v.at[buf, pl.ds(off, _IDX_SPLIT[j])],
          sems[buf])

    def fire(buf, s_loc):
      for j in range(2):
        descr(buf, s_loc, j).start()

    def drain(buf, s_loc):
      for j in range(2):
        descr(buf, s_loc, j).wait()

    def accumulate(buf):
      def body(r, accs):
        a0, a1, a2, a3 = accs
        for half in range(2):
          w = plsc.bitcast(rows_v[buf, r, pl.ds(half * _LANES, _LANES)],
                           jnp.int32)
          lo = plsc.bitcast(w << 16, jnp.float32)
          hi = plsc.bitcast(w, jnp.float32)
          if half == 0:
            a0, a2 = a0 + lo, a2 + hi
          else:
            a1, a3 = a1 + lo, a3 + hi
        return (a0, a1, a2, a3)
      zero = jnp.zeros((_LANES,), jnp.float32)
      return lax.fori_loop(0, _L, body, (zero,) * _CHUNKS, unroll=4)

    for h in range(_SPW // _QTR):
      base = wid * _SPW + h * _QTR
      pltpu.sync_copy(idx_hbm.at[pl.ds(base, _QTR)], idx_v)
      fire(0, 0)

      def step(i, carry):
        for bpar in range(2):
          s_loc = 2 * i + bpar
          nxt = s_loc + 1

          @pl.when(nxt < _QTR)
          def _():
            fire(1 - bpar, nxt)

          drain(bpar, s_loc)
          accs = accumulate(bpar)
          for c in range(_CHUNKS):
            out_v[s_loc, pl.ds(c * _LANES, _LANES)] = accs[c]
        return carry

      lax.fori_loop(0, _QTR // 2, step, 0)
      pltpu.sync_copy(out_v, out_hbm.at[pl.ds(base, _QTR)])

  return pool(table, idx2)


def _tc_linear(sums, w, b2):
  blk = 2048

  def body(x_ref, w_ref, b_ref, o_ref):
    o_ref[...] = lax.dot_general(
        x_ref[...], w_ref[...], (((1,), (1,)), ((), ())),
        preferred_element_type=jnp.float32) * (1.0 / _VOCAB) + b_ref[...]

  return pl.pallas_call(
      body,
      grid=(_B // blk,),
      in_specs=[
          pl.BlockSpec((blk, _EMBED), lambda i: (i, 0)),
          pl.BlockSpec((_IMG, _EMBED), lambda i: (0, 0)),
          pl.BlockSpec((1, _IMG), lambda i: (0, 0)),
      ],
      out_specs=pl.BlockSpec((blk, _IMG), lambda i: (i, 0)),
      out_shape=jax.ShapeDtypeStruct((_B, _IMG), jnp.float32),
  )(sums, w, b2)


def kernel(text_input, emb_table, W, b):
  ridx = _tc_idx_xform(text_input)
  packed = _tc_detile(emb_table.T)
  table_lin = packed.reshape(4 * _PACKED_ROWS, _PACKED_W)
  sums = _sc_pool(table_lin, ridx)
  return _tc_linear(sums, W, b.reshape(1, _IMG))


# speedup vs baseline: 1.2618x; 1.2618x over previous
import functools

import jax
import jax.numpy as jnp
from jax import lax
from jax.experimental import pallas as pl
from jax.experimental.pallas import tpu as pltpu
from jax.experimental.pallas import tpu_sc as plsc

_VOCAB = 1000000
_EMBED = 64
_IMG = 128
_B = 16384
_L = 200

_NC = 2
_NS = 16
_NW = _NC * _NS
_SPW = _B // _NW
_HALF = _SPW // 2
_IDX_PITCH = 256
_IDX_SPLIT = (96, 104)
_QTR = 128
_LANES = 16
_CHUNKS = _EMBED // _LANES
_PACKED_W = _EMBED // 2

_SB = 4096
_SB_LOG2 = _SB.bit_length() - 1
_NBLK = (_VOCAB + 4 * _SB - 1) // (4 * _SB)
_PACKED_ROWS = _NBLK * _SB


def _to_bf16_hi(bits):
  return (bits + 0x7FFF + ((bits >> 16) & 1)) >> 16


def _tc_detile(table_t):
  def body(x0_ref, x1_ref, x2_ref, x3_ref, o_ref):
    for j, x_ref in enumerate((x0_ref, x1_ref, x2_ref, x3_ref)):
      x = x_ref[...]
      lo = lax.bitcast_convert_type(x[:_PACKED_W, :], jnp.int32)
      hi = lax.bitcast_convert_type(x[_PACKED_W:, :], jnp.int32)
      w = (_to_bf16_hi(hi) << 16) | (_to_bf16_hi(lo) & 0xFFFF)
      wf = lax.bitcast_convert_type(w, jnp.float32)
      o_ref[:, pl.ds(j * _PACKED_W, _PACKED_W)] = jnp.transpose(wf)

  nb_in = (_VOCAB + _SB - 1) // _SB - 1

  def imap(j):
    return lambda i: (0, jnp.minimum(4 * i + j, nb_in))

  return pl.pallas_call(
      body,
      grid=(_NBLK,),
      in_specs=[pl.BlockSpec((_EMBED, _SB), imap(j)) for j in range(4)],
      out_specs=pl.BlockSpec((_SB, 4 * _PACKED_W), lambda i: (i, 0)),
      out_shape=jax.ShapeDtypeStruct((_PACKED_ROWS, 4 * _PACKED_W),
                                     jnp.float32),
  )(table_t, table_t, table_t, table_t)


def _tc_idx_xform(text):
  blk = 2048

  def body(x_ref, o_ref):
    v = x_ref[...]
    r = ((v & ~(4 * _SB - 1)) + ((v & (_SB - 1)) << 2)
         + ((v >> _SB_LOG2) & 3))
    pad = jnp.zeros((blk, _IDX_PITCH - _L), jnp.int32)
    o_ref[...] = jnp.concatenate([r, pad], axis=1)

  return pl.pallas_call(
      body,
      grid=(_B // blk,),
      in_specs=[pl.BlockSpec((blk, _L), lambda i: (i, 0))],
      out_specs=pl.BlockSpec((blk, _IDX_PITCH), lambda i: (i, 0)),
      out_shape=jax.ShapeDtypeStruct((_B, _IDX_PITCH), jnp.int32),
  )(text)


def _sc_pool(table, idx2):
  mesh = plsc.VectorSubcoreMesh(core_axis_name="c", subcore_axis_name="s")

  @functools.partial(
      pl.kernel,
      out_type=jax.ShapeDtypeStruct((_B, _EMBED), jnp.float32),
      mesh=mesh,
      compiler_params=pltpu.CompilerParams(use_tc_tiling_on_sc=False,
                                           needs_layout_passes=False),
      scratch_types=[
          pltpu.VMEM((_QTR, _IDX_PITCH), jnp.int32),
          pltpu.VMEM((4, _L, _PACKED_W), jnp.float32),
          pltpu.VMEM((_QTR, _EMBED), jnp.float32),
          pltpu.SemaphoreType.DMA,
          pltpu.SemaphoreType.DMA,
          pltpu.SemaphoreType.DMA,
          pltpu.SemaphoreType.DMA,
      ],
  )
  def pool(table_hbm, idx_hbm, out_hbm, idx_v, rows_v, out_v,
           sem0, sem1, sem2, sem3):
    wid = lax.axis_index("s") * _NC + lax.axis_index("c")
    sems = (sem0, sem1, sem2, sem3)

    def descr(buf, s_loc, j):
      off = j * _IDX_SPLIT[0]
      return pltpu.make_async_copy(
          table_hbm.at[idx_v.at[s_loc, pl.ds(off, _IDX_SPLIT[j])]],
          rows_v.at[buf, pl.ds(off, _IDX_SPLIT[j])],
          sems[buf])

    def fire(buf, s_loc):
      for j in range(2):
        descr(buf, s_loc, j).start()

    def drain(buf, s_loc):
      for j in range(2):
        descr(buf, s_loc, j).wait()

    def accumulate(buf):
      def body(r, accs):
        a0, a1, a2, a3 = accs
        for half in range(2):
          w = plsc.bitcast(rows_v[buf, r, pl.ds(half * _LANES, _LANES)],
                           jnp.int32)
          lo = plsc.bitcast(w << 16, jnp.float32)
          hi = plsc.bitcast(w, jnp.float32)
          if half == 0:
            a0, a2 = a0 + lo, a2 + hi
          else:
            a1, a3 = a1 + lo, a3 + hi
        return (a0, a1, a2, a3)
      zero = jnp.zeros((_LANES,), jnp.float32)
      return lax.fori_loop(0, _L, body, (zero,) * _CHUNKS, unroll=4)

    for h in range(_SPW // _QTR):
      base = wid * _SPW + h * _QTR
      pltpu.sync_copy(idx_hbm.at[pl.ds(base, _QTR)], idx_v)
      for p in range(3):
        fire(p, p)

      def step(i, carry):
        for bpar in range(4):
          s_loc = 4 * i + bpar
          nxt = s_loc + 3

          @pl.when(nxt < _QTR)
          def _():
            fire((bpar + 3) % 4, nxt)

          drain(bpar, s_loc)
          accs = accumulate(bpar)
          for c in range(_CHUNKS):
            out_v[s_loc, pl.ds(c * _LANES, _LANES)] = accs[c]
        return carry

      lax.fori_loop(0, _QTR // 4, step, 0)
      pltpu.sync_copy(out_v, out_hbm.at[pl.ds(base, _QTR)])

  return pool(table, idx2)


def _tc_linear(sums, w, b2):
  blk = 2048

  def body(x_ref, w_ref, b_ref, o_ref):
    o_ref[...] = lax.dot_general(
        x_ref[...], w_ref[...], (((1,), (1,)), ((), ())),
        preferred_element_type=jnp.float32) * (1.0 / _VOCAB) + b_ref[...]

  return pl.pallas_call(
      body,
      grid=(_B // blk,),
      in_specs=[
          pl.BlockSpec((blk, _EMBED), lambda i: (i, 0)),
          pl.BlockSpec((_IMG, _EMBED), lambda i: (0, 0)),
          pl.BlockSpec((1, _IMG), lambda i: (0, 0)),
      ],
      out_specs=pl.BlockSpec((blk, _IMG), lambda i: (i, 0)),
      out_shape=jax.ShapeDtypeStruct((_B, _IMG), jnp.float32),
  )(sums, w, b2)


def kernel(text_input, emb_table, W, b):
  ridx = _tc_idx_xform(text_input)
  packed = _tc_detile(emb_table.T)
  table_lin = packed.reshape(4 * _PACKED_ROWS, _PACKED_W)
  sums = _sc_pool(table_lin, ridx)
  return _tc_linear(sums, W, b.reshape(1, _IMG))


# speedup vs baseline: 1.3493x; 1.0693x over previous
import functools

import jax
import jax.numpy as jnp
from jax import lax
from jax.experimental import pallas as pl
from jax.experimental.pallas import tpu as pltpu
from jax.experimental.pallas import tpu_sc as plsc

_VOCAB = 1000000
_EMBED = 64
_IMG = 128
_B = 16384
_L = 200

_NC = 2
_NS = 16
_NW = _NC * _NS
_SPW = _B // _NW
_HALF = _SPW // 2
_IDX_PITCH = 256
_IDX_SPLIT = (96, 104)
_QTR = 128
_LANES = 16
_CHUNKS = _EMBED // _LANES
_PACKED_W = _EMBED // 2

_SB = 4096
_SB_LOG2 = _SB.bit_length() - 1
_NBLK = (_VOCAB + 4 * _SB - 1) // (4 * _SB)
_PACKED_ROWS = _NBLK * _SB


def _to_bf16_hi(bits):
  return (bits + 0x7FFF + ((bits >> 16) & 1)) >> 16


def _tc_detile(table_t):
  def body(x0_ref, x1_ref, x2_ref, x3_ref, o_ref):
    for j, x_ref in enumerate((x0_ref, x1_ref, x2_ref, x3_ref)):
      x = x_ref[...]
      lo = lax.bitcast_convert_type(x[:_PACKED_W, :], jnp.int32)
      hi = lax.bitcast_convert_type(x[_PACKED_W:, :], jnp.int32)
      w = (_to_bf16_hi(hi) << 16) | (_to_bf16_hi(lo) & 0xFFFF)
      wf = lax.bitcast_convert_type(w, jnp.float32)
      o_ref[:, pl.ds(j * _PACKED_W, _PACKED_W)] = jnp.transpose(wf)

  nb_in = (_VOCAB + _SB - 1) // _SB - 1

  def imap(j):
    return lambda i: (0, jnp.minimum(4 * i + j, nb_in))

  return pl.pallas_call(
      body,
      grid=(_NBLK,),
      in_specs=[pl.BlockSpec((_EMBED, _SB), imap(j)) for j in range(4)],
      out_specs=pl.BlockSpec((_SB, 4 * _PACKED_W), lambda i: (i, 0)),
      out_shape=jax.ShapeDtypeStruct((_PACKED_ROWS, 4 * _PACKED_W),
                                     jnp.float32),
  )(table_t, table_t, table_t, table_t)


def _tc_idx_xform(text):
  blk = 2048

  def body(x_ref, o_ref):
    v = x_ref[...]
    r = ((v & ~(4 * _SB - 1)) + ((v & (_SB - 1)) << 2)
         + ((v >> _SB_LOG2) & 3))
    pad = jnp.zeros((blk, _IDX_PITCH - _L), jnp.int32)
    o_ref[...] = jnp.concatenate([r, pad], axis=1)

  return pl.pallas_call(
      body,
      grid=(_B // blk,),
      in_specs=[pl.BlockSpec((blk, _L), lambda i: (i, 0))],
      out_specs=pl.BlockSpec((blk, _IDX_PITCH), lambda i: (i, 0)),
      out_shape=jax.ShapeDtypeStruct((_B, _IDX_PITCH), jnp.int32),
  )(text)


def _sc_pool(table, idx2):
  mesh = plsc.VectorSubcoreMesh(core_axis_name="c", subcore_axis_name="s")

  @functools.partial(
      pl.kernel,
      out_type=jax.ShapeDtypeStruct((_B, _EMBED), jnp.float32),
      mesh=mesh,
      compiler_params=pltpu.CompilerParams(use_tc_tiling_on_sc=False,
                                           needs_layout_passes=False),
      scratch_types=[
          pltpu.VMEM((_QTR, _IDX_PITCH), jnp.int32),
          pltpu.VMEM((8, _L, _PACKED_W), jnp.float32),
          pltpu.VMEM((_QTR, _EMBED), jnp.float32),
      ] + [pltpu.SemaphoreType.DMA] * 8,
  )
  def pool(table_hbm, idx_hbm, out_hbm, idx_v, rows_v, out_v, *sems):
    wid = lax.axis_index("s") * _NC + lax.axis_index("c")

    def descr(buf, s_loc, j):
      off = j * _IDX_SPLIT[0]
      return pltpu.make_async_copy(
          table_hbm.at[idx_v.at[s_loc, pl.ds(off, _IDX_SPLIT[j])]],
          rows_v.at[buf, pl.ds(off, _IDX_SPLIT[j])],
          sems[buf])

    def fire(buf, s_loc):
      for j in range(2):
        descr(buf, s_loc, j).start()

    def drain(buf, s_loc):
      for j in range(2):
        descr(buf, s_loc, j).wait()

    def accumulate(buf):
      def body(r, accs):
        a0, a1, a2, a3 = accs
        for half in range(2):
          w = plsc.bitcast(rows_v[buf, r, pl.ds(half * _LANES, _LANES)],
                           jnp.int32)
          lo = plsc.bitcast(w << 16, jnp.float32)
          hi = plsc.bitcast(w, jnp.float32)
          if half == 0:
            a0, a2 = a0 + lo, a2 + hi
          else:
            a1, a3 = a1 + lo, a3 + hi
        return (a0, a1, a2, a3)
      zero = jnp.zeros((_LANES,), jnp.float32)
      return lax.fori_loop(0, _L, body, (zero,) * _CHUNKS, unroll=4)

    for h in range(_SPW // _QTR):
      base = wid * _SPW + h * _QTR
      pltpu.sync_copy(idx_hbm.at[pl.ds(base, _QTR)], idx_v)
      for p in range(7):
        fire(p, p)

      def step(i, carry):
        for bpar in range(8):
          s_loc = 8 * i + bpar
          nxt = s_loc + 7

          @pl.when(nxt < _QTR)
          def _():
            fire((bpar + 7) % 8, nxt)

          drain(bpar, s_loc)
          accs = accumulate(bpar)
          for c in range(_CHUNKS):
            out_v[s_loc, pl.ds(c * _LANES, _LANES)] = accs[c]
        return carry

      lax.fori_loop(0, _QTR // 8, step, 0)
      pltpu.sync_copy(out_v, out_hbm.at[pl.ds(base, _QTR)])

  return pool(table, idx2)


def _tc_linear(sums, w, b2):
  blk = 2048

  def body(x_ref, w_ref, b_ref, o_ref):
    o_ref[...] = lax.dot_general(
        x_ref[...], w_ref[...], (((1,), (1,)), ((), ())),
        preferred_element_type=jnp.float32) * (1.0 / _VOCAB) + b_ref[...]

  return pl.pallas_call(
      body,
      grid=(_B // blk,),
      in_specs=[
          pl.BlockSpec((blk, _EMBED), lambda i: (i, 0)),
          pl.BlockSpec((_IMG, _EMBED), lambda i: (0, 0)),
          pl.BlockSpec((1, _IMG), lambda i: (0, 0)),
      ],
      out_specs=pl.BlockSpec((blk, _IMG), lambda i: (i, 0)),
      out_shape=jax.ShapeDtypeStruct((_B, _IMG), jnp.float32),
  )(sums, w, b2)


def kernel(text_input, emb_table, W, b):
  ridx = _tc_idx_xform(text_input)
  packed = _tc_detile(emb_table.T)
  table_lin = packed.reshape(4 * _PACKED_ROWS, _PACKED_W)
  sums = _sc_pool(table_lin, ridx)
  return _tc_linear(sums, W, b.reshape(1, _IMG))


# speedup vs baseline: 1.3656x; 1.0121x over previous
import functools

import jax
import jax.numpy as jnp
from jax import lax
from jax.experimental import pallas as pl
from jax.experimental.pallas import tpu as pltpu
from jax.experimental.pallas import tpu_sc as plsc

_VOCAB = 1000000
_EMBED = 64
_IMG = 128
_B = 16384
_L = 200

_NC = 2
_NS = 16
_NW = _NC * _NS
_SPW = _B // _NW
_HALF = _SPW // 2
_IDX_PITCH = 256
_IDX_SPLIT = (96, 104)
_QTR = 128
_LANES = 16
_CHUNKS = _EMBED // _LANES
_PACKED_W = _EMBED // 2

_SB = 8192
_SB_LOG2 = _SB.bit_length() - 1
_NBLK = (_VOCAB + 4 * _SB - 1) // (4 * _SB)
_PACKED_ROWS = _NBLK * _SB


def _to_bf16_hi(bits):
  return (bits + 0x7FFF + ((bits >> 16) & 1)) >> 16


def _tc_detile(table_t):
  def body(x0_ref, x1_ref, x2_ref, x3_ref, o_ref):
    for j, x_ref in enumerate((x0_ref, x1_ref, x2_ref, x3_ref)):
      x = x_ref[...]
      lo = lax.bitcast_convert_type(x[:_PACKED_W, :], jnp.int32)
      hi = lax.bitcast_convert_type(x[_PACKED_W:, :], jnp.int32)
      w = (_to_bf16_hi(hi) << 16) | (_to_bf16_hi(lo) & 0xFFFF)
      wf = lax.bitcast_convert_type(w, jnp.float32)
      o_ref[:, pl.ds(j * _PACKED_W, _PACKED_W)] = jnp.transpose(wf)

  nb_in = (_VOCAB + _SB - 1) // _SB - 1

  def imap(j):
    return lambda i: (0, jnp.minimum(4 * i + j, nb_in))

  return pl.pallas_call(
      body,
      grid=(_NBLK,),
      in_specs=[pl.BlockSpec((_EMBED, _SB), imap(j)) for j in range(4)],
      out_specs=pl.BlockSpec((_SB, 4 * _PACKED_W), lambda i: (i, 0)),
      out_shape=jax.ShapeDtypeStruct((_PACKED_ROWS, 4 * _PACKED_W),
                                     jnp.float32),
  )(table_t, table_t, table_t, table_t)


def _tc_idx_xform(text):
  blk = 2048

  def body(x_ref, o_ref):
    v = x_ref[...]
    r = ((v & ~(4 * _SB - 1)) + ((v & (_SB - 1)) << 2)
         + ((v >> _SB_LOG2) & 3))
    pad = jnp.zeros((blk, _IDX_PITCH - _L), jnp.int32)
    o_ref[...] = jnp.concatenate([r, pad], axis=1)

  return pl.pallas_call(
      body,
      grid=(_B // blk,),
      in_specs=[pl.BlockSpec((blk, _L), lambda i: (i, 0))],
      out_specs=pl.BlockSpec((blk, _IDX_PITCH), lambda i: (i, 0)),
      out_shape=jax.ShapeDtypeStruct((_B, _IDX_PITCH), jnp.int32),
  )(text)


def _sc_pool(table, idx2):
  mesh = plsc.VectorSubcoreMesh(core_axis_name="c", subcore_axis_name="s")

  @functools.partial(
      pl.kernel,
      out_type=jax.ShapeDtypeStruct((_B, _EMBED), jnp.float32),
      mesh=mesh,
      compiler_params=pltpu.CompilerParams(use_tc_tiling_on_sc=False,
                                           needs_layout_passes=False),
      scratch_types=[
          pltpu.VMEM((_QTR, _IDX_PITCH), jnp.int32),
          pltpu.VMEM((8, _L, _PACKED_W), jnp.float32),
          pltpu.VMEM((_QTR, _EMBED), jnp.float32),
      ] + [pltpu.SemaphoreType.DMA] * 8,
  )
  def pool(table_hbm, idx_hbm, out_hbm, idx_v, rows_v, out_v, *sems):
    wid = lax.axis_index("s") * _NC + lax.axis_index("c")

    def descr(buf, s_loc, j):
      off = j * _IDX_SPLIT[0]
      return pltpu.make_async_copy(
          table_hbm.at[idx_v.at[s_loc, pl.ds(off, _IDX_SPLIT[j])]],
          rows_v.at[buf, pl.ds(off, _IDX_SPLIT[j])],
          sems[buf])

    def fire(buf, s_loc):
      for j in range(2):
        descr(buf, s_loc, j).start()

    def drain(buf, s_loc):
      for j in range(2):
        descr(buf, s_loc, j).wait()

    def accumulate(buf):
      def body(r, accs):
        a0, a1, a2, a3 = accs
        for half in range(2):
          w = plsc.bitcast(rows_v[buf, r, pl.ds(half * _LANES, _LANES)],
                           jnp.int32)
          lo = plsc.bitcast(w << 16, jnp.float32)
          hi = plsc.bitcast(w, jnp.float32)
          if half == 0:
            a0, a2 = a0 + lo, a2 + hi
          else:
            a1, a3 = a1 + lo, a3 + hi
        return (a0, a1, a2, a3)
      zero = jnp.zeros((_LANES,), jnp.float32)
      return lax.fori_loop(0, _L, body, (zero,) * _CHUNKS, unroll=4)

    for h in range(_SPW // _QTR):
      base = wid * _SPW + h * _QTR
      pltpu.sync_copy(idx_hbm.at[pl.ds(base, _QTR)], idx_v)
      for p in range(7):
        fire(p, p)

      def step(i, carry):
        for bpar in range(8):
          s_loc = 8 * i + bpar
          nxt = s_loc + 7

          @pl.when(nxt < _QTR)
          def _():
            fire((bpar + 7) % 8, nxt)

          drain(bpar, s_loc)
          accs = accumulate(bpar)
          for c in range(_CHUNKS):
            out_v[s_loc, pl.ds(c * _LANES, _LANES)] = accs[c]
        return carry

      lax.fori_loop(0, _QTR // 8, step, 0)
      pltpu.sync_copy(out_v, out_hbm.at[pl.ds(base, _QTR)])

  return pool(table, idx2)


def _tc_linear(sums, w, b2):
  blk = 2048

  def body(x_ref, w_ref, b_ref, o_ref):
    o_ref[...] = lax.dot_general(
        x_ref[...], w_ref[...], (((1,), (1,)), ((), ())),
        preferred_element_type=jnp.float32) * (1.0 / _VOCAB) + b_ref[...]

  return pl.pallas_call(
      body,
      grid=(_B // blk,),
      in_specs=[
          pl.BlockSpec((blk, _EMBED), lambda i: (i, 0)),
          pl.BlockSpec((_IMG, _EMBED), lambda i: (0, 0)),
          pl.BlockSpec((1, _IMG), lambda i: (0, 0)),
      ],
      out_specs=pl.BlockSpec((blk, _IMG), lambda i: (i, 0)),
      out_shape=jax.ShapeDtypeStruct((_B, _IMG), jnp.float32),
  )(sums, w, b2)


def kernel(text_input, emb_table, W, b):
  ridx = _tc_idx_xform(text_input)
  packed = _tc_detile(emb_table.T)
  table_lin = packed.reshape(4 * _PACKED_ROWS, _PACKED_W)
  sums = _sc_pool(table_lin, ridx)
  return _tc_linear(sums, W, b.reshape(1, _IMG))
